# baseline (speedup 1.0000x reference)
import functools

import jax
import jax.numpy as jnp
from jax.experimental import pallas as pl
from jax.experimental.pallas import tpu as pltpu

S, D = 2048, 1024
H, KV, HD = 16, 8, 64
E, TOPK, F = 8, 2, 1024
L = 2
EPS = 1e-6
THETA = 10000.0
HH = HD // 2

QW = H * HH
KW = KV * HH


def _rms(x, w):
    return x * jax.lax.rsqrt(jnp.mean(x * x, axis=-1, keepdims=True) + EPS) * w


def _qkv_body(h_ref, ln1_ref, wq_ref, wk_ref, wv_ref, cq_ref, sq_ref,
              ck_ref, sk_ref, q_out, k_out, v_out):
    r = _rms(h_ref[...], ln1_ref[...])
    q = jnp.dot(r, wq_ref[...], preferred_element_type=jnp.float32)
    k = jnp.dot(r, wk_ref[...], preferred_element_type=jnp.float32)
    v = jnp.dot(r, wv_ref[...], preferred_element_type=jnp.float32)
    qr = jnp.concatenate([-q[:, QW:], q[:, :QW]], axis=1)
    kr = jnp.concatenate([-k[:, KW:], k[:, :KW]], axis=1)
    q_out[...] = q * cq_ref[...] + qr * sq_ref[...]
    k_out[...] = k * ck_ref[...] + kr * sk_ref[...]
    v_out[...] = v


BSQ = 512
NSB = S // BSQ


def _qkv_call(h, ln1, wq_p, wk_p, wv, cq, sq, ck, sk):
    return pl.pallas_call(
        _qkv_body,
        grid=(NSB,),
        in_specs=[
            pl.BlockSpec((BSQ, D), lambda i: (i, 0)),
            pl.BlockSpec((D,), lambda i: (0,)),
            pl.BlockSpec((D, 2 * QW), lambda i: (0, 0)),
            pl.BlockSpec((D, 2 * KW), lambda i: (0, 0)),
            pl.BlockSpec((D, KV * HD), lambda i: (0, 0)),
            pl.BlockSpec((BSQ, 2 * QW), lambda i: (i, 0)),
            pl.BlockSpec((BSQ, 2 * QW), lambda i: (i, 0)),
            pl.BlockSpec((BSQ, 2 * KW), lambda i: (i, 0)),
            pl.BlockSpec((BSQ, 2 * KW), lambda i: (i, 0)),
        ],
        out_specs=[
            pl.BlockSpec((BSQ, 2 * QW), lambda i: (i, 0)),
            pl.BlockSpec((BSQ, 2 * KW), lambda i: (i, 0)),
            pl.BlockSpec((BSQ, KV * HD), lambda i: (i, 0)),
        ],
        out_shape=[
            jax.ShapeDtypeStruct((S, 2 * QW), jnp.float32),
            jax.ShapeDtypeStruct((S, 2 * KW), jnp.float32),
            jax.ShapeDtypeStruct((S, KV * HD), jnp.float32),
        ],
    )(h, ln1, wq_p, wk_p, wv, cq, sq, ck, sk)


BQ = 256
NQB = S // BQ


def _attn_body(q_ref, k_ref, v_ref, o_ref):
    qb_i = pl.program_id(0)
    q = q_ref[...]
    k = k_ref[...]
    v = v_ref[...]
    rows = jax.lax.broadcasted_iota(jnp.int32, (BQ, S), 0) + qb_i * BQ
    cols = jax.lax.broadcasted_iota(jnp.int32, (BQ, S), 1)
    bias = jnp.where(cols <= rows, 0.0, -1e9)
    for h in range(H):
        j = h // 2
        qh = jnp.concatenate(
            [q[:, h * HH:(h + 1) * HH], q[:, QW + h * HH:QW + (h + 1) * HH]],
            axis=1)
        kh = jnp.concatenate(
            [k[:, j * HH:(j + 1) * HH], k[:, KW + j * HH:KW + (j + 1) * HH]],
            axis=1)
        s = jnp.dot(qh, kh.T, preferred_element_type=jnp.float32)
        s = s * (1.0 / (HD ** 0.5)) + bias
        p = jax.nn.softmax(s, axis=-1)
        o_ref[:, h * HD:(h + 1) * HD] = jnp.dot(
            p, v[:, j * HD:(j + 1) * HD], preferred_element_type=jnp.float32)


def _attn_call(q, k, v):
    return pl.pallas_call(
        _attn_body,
        grid=(NQB,),
        in_specs=[
            pl.BlockSpec((BQ, 2 * QW), lambda qb: (qb, 0)),
            pl.BlockSpec((S, 2 * KW), lambda qb: (0, 0)),
            pl.BlockSpec((S, KV * HD), lambda qb: (0, 0)),
        ],
        out_specs=pl.BlockSpec((BQ, H * HD), lambda qb: (qb, 0)),
        out_shape=jax.ShapeDtypeStruct((S, H * HD), jnp.float32),
        compiler_params=pltpu.CompilerParams(
            vmem_limit_bytes=100 * 1024 * 1024),
    )(q, k, v)


def _wo_router_body(h_ref, o_ref, wo_ref, ln2_ref, wg_ref,
                    h2_out, r2_out, ti_out, tw_out):
    h2 = h_ref[...] + jnp.dot(o_ref[...], wo_ref[...],
                              preferred_element_type=jnp.float32)
    h2_out[...] = h2
    r2 = _rms(h2, ln2_ref[...])
    r2_out[...] = r2
    logits = jnp.dot(r2, wg_ref[...], preferred_element_type=jnp.float32)
    probs = jax.nn.softmax(logits, axis=-1)
    idx = jax.lax.broadcasted_iota(jnp.int32, (BSQ, E), 1)
    m1 = jnp.max(probs, axis=-1, keepdims=True)
    i1 = jnp.min(jnp.where(probs == m1, idx, E), axis=-1, keepdims=True)
    oh1 = idx == i1
    rest = jnp.where(oh1, -jnp.inf, probs)
    m2 = jnp.max(rest, axis=-1, keepdims=True)
    i2 = jnp.min(jnp.where(rest == m2, idx, E), axis=-1, keepdims=True)
    oh2 = idx == i2
    denom = m1 + m2
    ti_out[...] = jnp.concatenate([i1, i2], axis=1)
    tw_out[...] = jnp.concatenate([m1 / denom, m2 / denom], axis=1)


def _wo_router_call(h, o, wo, ln2, wg):
    return pl.pallas_call(
        _wo_router_body,
        grid=(NSB,),
        in_specs=[
            pl.BlockSpec((BSQ, D), lambda i: (i, 0)),
            pl.BlockSpec((BSQ, H * HD), lambda i: (i, 0)),
            pl.BlockSpec((H * HD, D), lambda i: (0, 0)),
            pl.BlockSpec((D,), lambda i: (0,)),
            pl.BlockSpec((D, E), lambda i: (0, 0)),
        ],
        out_specs=[
            pl.BlockSpec((BSQ, D), lambda i: (i, 0)),
            pl.BlockSpec((BSQ, D), lambda i: (i, 0)),
            pl.BlockSpec((BSQ, TOPK), lambda i: (i, 0)),
            pl.BlockSpec((BSQ, TOPK), lambda i: (i, 0)),
        ],
        out_shape=[
            jax.ShapeDtypeStruct((S, D), jnp.float32),
            jax.ShapeDtypeStruct((S, D), jnp.float32),
            jax.ShapeDtypeStruct((S, TOPK), jnp.int32),
            jax.ShapeDtypeStruct((S, TOPK), jnp.float32),
        ],
    )(h, o, wo, ln2, wg)


A = TOPK * S
BLK = 256
NB = A // BLK
G = NB + E - 1

SC_NC, SC_NS = 2, 16
SC_NW = SC_NC * SC_NS


def _make_sc_gather(V, B):
    from jax.experimental.pallas import tpu_sc as plsc
    b_per_w = B // SC_NW
    CH = 32
    nch = b_per_w // CH
    mesh = plsc.VectorSubcoreMesh(core_axis_name="c", subcore_axis_name="s",
                                  num_cores=SC_NC)

    @functools.partial(
        pl.kernel, mesh=mesh,
        out_type=jax.ShapeDtypeStruct((B, D), jnp.float32),
        scratch_types=[
            pltpu.VMEM((CH,), jnp.int32),
            pltpu.VMEM((CH, D), jnp.float32),
            pltpu.SemaphoreType.DMA,
        ],
    )
    def gather_k(table_hbm, idx_hbm, out_hbm, idx_v, rows_v, sem):
        wid = jax.lax.axis_index("s") * SC_NC + jax.lax.axis_index("c")
        base = wid * b_per_w
        for c in range(nch):
            pltpu.sync_copy(idx_hbm.at[pl.ds(base + c * CH, CH)], idx_v)
            pltpu.async_copy(table_hbm.at[idx_v], rows_v, sem).wait()
            pltpu.sync_copy(rows_v, out_hbm.at[pl.ds(base + c * CH, CH)])

    return gather_k


_sc_gather_cache = {}


def _sc_gather(table, idx):
    key = (table.shape[0], idx.shape[0])
    if key not in _sc_gather_cache:
        _sc_gather_cache[key] = _make_sc_gather(*key)
    return _sc_gather_cache[key](table, idx)


def _gmm_body(b_arr, e_arr, val_arr, off_arr,
              x_ref, ws_ref, w1_ref, w3_ref, w2_ref, out_ref):
    g = pl.program_id(0)
    b = b_arr[g]
    e = e_arr[g]
    start = jnp.maximum(off_arr[e], b * BLK)
    end = jnp.minimum(off_arr[e + 1], (b + 1) * BLK)
    rows = jax.lax.broadcasted_iota(jnp.int32, (BLK, 1), 0) + b * BLK
    mask = (rows >= start) & (rows < end) & (val_arr[g] > 0)
    x = x_ref[...]
    a = jnp.dot(x, w1_ref[0], preferred_element_type=jnp.float32)
    bb = jnp.dot(x, w3_ref[0], preferred_element_type=jnp.float32)
    y = jnp.dot(a * jax.nn.sigmoid(a) * bb, w2_ref[0],
                preferred_element_type=jnp.float32)
    y = jnp.where(mask, y * ws_ref[...], 0.0)
    b_prev = b_arr[jnp.maximum(g - 1, 0)]
    first = (g == 0) | (b != b_prev)

    @pl.when(first)
    def _():
        out_ref[...] = y

    @pl.when(jnp.logical_not(first))
    def _():
        out_ref[...] += y


def _gmm_call(xg, ws, w1, w3, w2, b_arr, e_arr, val_arr, off):
    grid_spec = pltpu.PrefetchScalarGridSpec(
        num_scalar_prefetch=4,
        grid=(G,),
        in_specs=[
            pl.BlockSpec((BLK, D), lambda g, bs, es, vs, os: (bs[g], 0)),
            pl.BlockSpec((BLK, 1), lambda g, bs, es, vs, os: (bs[g], 0)),
            pl.BlockSpec((1, D, F), lambda g, bs, es, vs, os: (es[g], 0, 0)),
            pl.BlockSpec((1, D, F), lambda g, bs, es, vs, os: (es[g], 0, 0)),
            pl.BlockSpec((1, F, D), lambda g, bs, es, vs, os: (es[g], 0, 0)),
        ],
        out_specs=pl.BlockSpec((BLK, D), lambda g, bs, es, vs, os: (bs[g], 0)),
    )
    return pl.pallas_call(
        _gmm_body,
        grid_spec=grid_spec,
        out_shape=jax.ShapeDtypeStruct((A, D), jnp.float32),
        compiler_params=pltpu.CompilerParams(
            vmem_limit_bytes=100 * 1024 * 1024),
    )(b_arr, e_arr, val_arr, off, xg, ws, w1, w3, w2)


def _combine_body(h2_ref, ya_ref, yb_ref, out_ref):
    out_ref[...] = h2_ref[...] + ya_ref[...] + yb_ref[...]


def _combine_call(h2, yg):
    return pl.pallas_call(
        _combine_body,
        grid=(NSB,),
        in_specs=[
            pl.BlockSpec((BSQ, D), lambda i: (i, 0)),
            pl.BlockSpec((BSQ, D), lambda i: (i, 0)),
            pl.BlockSpec((BSQ, D), lambda i: (S // BSQ + i, 0)),
        ],
        out_specs=pl.BlockSpec((BSQ, D), lambda i: (i, 0)),
        out_shape=jax.ShapeDtypeStruct((S, D), jnp.float32),
    )(h2, yg, yg)


def _route(ti, tw):
    eid = jnp.concatenate([ti[:, 0], ti[:, 1]])
    wts = jnp.concatenate([tw[:, 0], tw[:, 1]])
    sortidx = jnp.argsort(eid).astype(jnp.int32)
    tok_s = (sortidx % S).astype(jnp.int32)
    w_s = wts[sortidx][:, None]
    counts = jnp.bincount(eid, length=E)
    off = jnp.concatenate([jnp.zeros((1,), jnp.int32),
                           jnp.cumsum(counts).astype(jnp.int32)])
    invp = jnp.zeros((A,), jnp.int32).at[sortidx].set(
        jnp.arange(A, dtype=jnp.int32))
    lo = off[:E] // BLK
    hi = (off[1:] - 1) // BLK
    nb = jnp.where(counts > 0, hi - lo + 1, 0)
    gof = jnp.concatenate([jnp.zeros((1,), jnp.int32),
                           jnp.cumsum(nb).astype(jnp.int32)])
    total = gof[E]
    gidx = jnp.arange(G, dtype=jnp.int32)
    e_of_g = jnp.clip(jnp.searchsorted(gof, gidx, side='right') - 1, 0, E - 1)
    b_of_g = lo[e_of_g] + gidx - gof[e_of_g]
    validg = gidx < total
    b_arr = jnp.where(validg, b_of_g, NB - 1).astype(jnp.int32)
    e_arr = jnp.where(validg, e_of_g, E - 1).astype(jnp.int32)
    val_arr = validg.astype(jnp.int32)
    return tok_s, w_s, off, invp, b_arr, e_arr, val_arr


def _final_body(h_ref, w_ref, out_ref):
    out_ref[...] = _rms(h_ref[...], w_ref[...])


def _final_call(h, w):
    return pl.pallas_call(
        _final_body,
        out_shape=jax.ShapeDtypeStruct((S, D), jnp.float32),
    )(h, w)


def _col_perm_q():
    import numpy as np
    n = np.arange(2 * QW)
    half, rest = n // QW, n % QW
    return (rest // HH) * HD + half * HH + rest % HH


def _col_perm_k():
    import numpy as np
    n = np.arange(2 * KW)
    half, rest = n // KW, n % KW
    return (rest // HH) * HD + half * HH + rest % HH


def _rope_tables():
    inv_freq = 1.0 / (THETA ** (jnp.arange(0, HD, 2).astype(jnp.float32) / HD))
    freqs = jnp.arange(S, dtype=jnp.float32)[:, None] * inv_freq[None, :]
    cosf, sinf = jnp.cos(freqs), jnp.sin(freqs)
    cq = jnp.tile(cosf, (1, 2 * QW // HH))
    sq = jnp.tile(sinf, (1, 2 * QW // HH))
    ck = jnp.tile(cosf, (1, 2 * KW // HH))
    sk = jnp.tile(sinf, (1, 2 * KW // HH))
    return cq, sq, ck, sk


@jax.jit
def _forward(x, params):
    cq, sq, ck, sk = _rope_tables()
    pq, pk = _col_perm_q(), _col_perm_k()
    h = x.reshape(S, D)
    for l in range(L):
        p = params['layer_%d' % l]
        q, k, v = _qkv_call(h, p['ln1'], p['wq'][:, pq], p['wk'][:, pk],
                            p['wv'], cq, sq, ck, sk)
        o = _attn_call(q, k, v)
        h2, r2, ti, tw = _wo_router_call(h, o, p['wo'], p['ln2'], p['wg'])
        tok_s, w_s, off, invp, b_arr, e_arr, val_arr = _route(ti, tw)
        xg = _sc_gather(r2, tok_s)
        y = _gmm_call(xg, w_s, p['w1'], p['w3'], p['w2'],
                      b_arr, e_arr, val_arr, off)
        yg = _sc_gather(y, invp)
        h = _combine_call(h2, yg)
    return _final_call(h, params['final_ln']).reshape(1, S, D)


def kernel(input_ids, params):
    return _forward(input_ids, params)
